# baseline (speedup 1.0000x reference)
import functools

import jax
import jax.numpy as jnp
from jax import lax
from jax.experimental import pallas as pl
from jax.experimental.pallas import tpu as pltpu
from jax.experimental.pallas import tpu_sc as plsc

N_NODES = 10000
N_EDGES = 160000
D = 256
DH = 128

NPAD = 10240
EPAD = 163840
CHUNK = 128
EPT = EPAD // 16
NCHUNKS = EPT // CHUNK
ROWS_PT = NPAD // 16


def _degrees_body(ei_ref, deg_ref, idx_v, zb_v, ones_v, hist_sh):
    c = lax.axis_index("c")
    s = lax.axis_index("s")

    z16 = jnp.zeros((16,), jnp.float32)
    o16 = jnp.ones((16,), jnp.float32)
    for j in range(CHUNK // 16):
        ones_v[pl.ds(16 * j, 16)] = o16

    def zfill(i, _):
        zb_v[pl.ds(16 * i, 16)] = z16
        return 0

    lax.fori_loop(0, ROWS_PT // 16, zfill, 0)

    pltpu.sync_copy(zb_v, hist_sh.at[pl.ds(s * ROWS_PT, ROWS_PT)])
    plsc.subcore_barrier()

    def chunk_step(k, _):
        base = s * EPT + k * CHUNK
        pltpu.sync_copy(ei_ref.at[c, pl.ds(base, CHUNK)], idx_v)
        pltpu.sync_copy(ones_v, hist_sh.at[idx_v], add=True)
        return 0

    lax.fori_loop(0, NCHUNKS, chunk_step, 0)
    plsc.subcore_barrier()

    pltpu.sync_copy(hist_sh.at[pl.ds(s * ROWS_PT, ROWS_PT)],
                    deg_ref.at[c, pl.ds(s * ROWS_PT, ROWS_PT)])


@functools.partial(
    pl.kernel,
    out_type=jax.ShapeDtypeStruct((2, NPAD), jnp.float32),
    mesh=plsc.VectorSubcoreMesh(core_axis_name="c", subcore_axis_name="s"),
    scratch_types=[
        pltpu.VMEM((CHUNK,), jnp.int32),
        pltpu.VMEM((ROWS_PT,), jnp.float32),
        pltpu.VMEM((CHUNK,), jnp.float32),
        pltpu.VMEM_SHARED((NPAD,), jnp.float32),
    ],
)
def _degrees(ei_ref, deg_ref, idx_v, zb_v, ones_v, hist_sh):
    _degrees_body(ei_ref, deg_ref, idx_v, zb_v, ones_v, hist_sh)


NBUF = 2


def _gs_body(h2_ref, gidx_ref, didx_ref, agg_ref,
             gi_a, di_a, di_b, rows_a, rows_b,
             dsem_a, dsem_b, gsem_a, gsem_b, ssem_a, ssem_b, agg_sh):
    c = lax.axis_index("c")
    s = lax.axis_index("s")

    z16 = jnp.zeros((16,), jnp.float32)

    def zrow(r, _):
        for j in range(DH // 16):
            rows_a[r, pl.ds(16 * j, 16)] = z16
        return 0

    lax.fori_loop(0, CHUNK, zrow, 0)

    def zcopy(i, _):
        pltpu.sync_copy(
            rows_a, agg_sh.at[pl.ds(s * ROWS_PT + i * CHUNK, CHUNK)])
        return 0

    lax.fori_loop(0, ROWS_PT // CHUNK, zcopy, 0)
    plsc.subcore_barrier()

    pltpu.sync_copy(gidx_ref.at[c, s], gi_a)

    def start_chunk(k, di_v, rows_v, dsem, gsem):
        base = s * EPT + k * CHUNK
        pltpu.async_copy(didx_ref.at[pl.ds(base, CHUNK)], di_v, dsem)
        pltpu.async_copy(h2_ref.at[gi_a.at[k]], rows_v, gsem)

    def start_scatter(k, di_v, rows_v, dsem, gsem, ssem):
        base = s * EPT + k * CHUNK
        pltpu.make_async_copy(didx_ref.at[pl.ds(base, CHUNK)], di_v,
                              dsem).wait()
        pltpu.make_async_copy(h2_ref.at[gi_a.at[k]], rows_v, gsem).wait()
        pltpu.async_copy(rows_v, agg_sh.at[di_v], ssem, add=True)

    def wait_scatter(di_v, rows_v, ssem):
        pltpu.make_async_copy(rows_v, agg_sh.at[di_v], ssem).wait()

    start_chunk(0, di_a, rows_a, dsem_a, gsem_a)
    start_chunk(1, di_b, rows_b, dsem_b, gsem_b)

    def pair_round(r, _):
        k = r * 2
        start_scatter(k, di_a, rows_a, dsem_a, gsem_a, ssem_a)
        start_scatter(k + 1, di_b, rows_b, dsem_b, gsem_b, ssem_b)
        wait_scatter(di_a, rows_a, ssem_a)

        @pl.when(k + 2 < NCHUNKS)
        def _():
            start_chunk(k + 2, di_a, rows_a, dsem_a, gsem_a)

        wait_scatter(di_b, rows_b, ssem_b)

        @pl.when(k + 3 < NCHUNKS)
        def _():
            start_chunk(k + 3, di_b, rows_b, dsem_b, gsem_b)

        return 0

    lax.fori_loop(0, NCHUNKS // 2, pair_round, 0)
    plsc.subcore_barrier()

    pltpu.sync_copy(agg_sh.at[pl.ds(s * ROWS_PT, ROWS_PT)],
                    agg_ref.at[c, pl.ds(s * ROWS_PT, ROWS_PT)])


@functools.partial(
    pl.kernel,
    out_type=jax.ShapeDtypeStruct((2, NPAD, DH), jnp.float32),
    mesh=plsc.VectorSubcoreMesh(core_axis_name="c", subcore_axis_name="s"),
    scratch_types=[
        pltpu.VMEM((NCHUNKS, CHUNK), jnp.int32),
        pltpu.VMEM((CHUNK,), jnp.int32),
        pltpu.VMEM((CHUNK,), jnp.int32),
        pltpu.VMEM((CHUNK, DH), jnp.float32),
        pltpu.VMEM((CHUNK, DH), jnp.float32),
        pltpu.SemaphoreType.DMA,
        pltpu.SemaphoreType.DMA,
        pltpu.SemaphoreType.DMA,
        pltpu.SemaphoreType.DMA,
        pltpu.SemaphoreType.DMA,
        pltpu.SemaphoreType.DMA,
        pltpu.VMEM_SHARED((NPAD, DH), jnp.float32),
    ],
)
def _gather_scatter(h2_ref, gidx_ref, didx_ref, agg_ref,
                    gi_a, di_a, di_b, rows_a, rows_b,
                    dsem_a, dsem_b, gsem_a, gsem_b, ssem_a, ssem_b, agg_sh):
    _gs_body(h2_ref, gidx_ref, didx_ref, agg_ref, gi_a, di_a, di_b,
             rows_a, rows_b, dsem_a, dsem_b, gsem_a, gsem_b, ssem_a, ssem_b,
             agg_sh)


def _matmul_scale_kernel(x_ref, w_ref, deg_ref, out_ref):
    deg = deg_ref[...]
    norm = jnp.where(deg > 0.0,
                     lax.rsqrt(jnp.maximum(deg, 1e-12)),
                     0.0)
    h = jnp.dot(x_ref[...], w_ref[...],
                preferred_element_type=jnp.float32) * norm
    out_ref[0] = h[:, :DH]
    out_ref[1] = h[:, DH:]


def _finalize_kernel(agg_ref, deg_ref, b_ref, out_ref):
    deg = deg_ref[...]
    norm = jnp.where(deg > 0.0,
                     lax.rsqrt(jnp.maximum(deg, 1e-12)),
                     0.0)
    agg = jnp.concatenate(
        [agg_ref[0, :N_NODES, :], agg_ref[1, :N_NODES, :]], axis=1)
    out_ref[...] = agg * norm + b_ref[...]


def kernel(inputs, edge_index, W, b):
    src = edge_index[0]
    dst = edge_index[1]

    pad = EPAD - N_EDGES
    pad_bin = jnp.full((pad,), NPAD - 1, jnp.int32)
    ei_deg = jnp.stack([jnp.concatenate([src, pad_bin]),
                        jnp.concatenate([dst, pad_bin])])
    pad_zero = jnp.zeros((pad,), jnp.int32)
    gidx = jnp.stack([jnp.concatenate([src, pad_zero]),
                      jnp.concatenate([src + N_NODES, pad_zero])])
    gidx = gidx.reshape(2, 16, NCHUNKS, CHUNK)
    didx = jnp.concatenate([dst, pad_bin])

    deg = _degrees(ei_deg)

    h2 = pl.pallas_call(
        _matmul_scale_kernel,
        out_shape=jax.ShapeDtypeStruct((2, N_NODES, DH), jnp.float32),
    )(inputs, W, deg[0, :N_NODES].reshape(N_NODES, 1))
    h2 = h2.reshape(2 * N_NODES, DH)

    agg = _gather_scatter(h2, gidx, didx)

    h1 = pl.pallas_call(
        _finalize_kernel,
        out_shape=jax.ShapeDtypeStruct((N_NODES, D), jnp.float32),
    )(agg, deg[1, :N_NODES].reshape(N_NODES, 1), b.reshape(1, D))
    return (h1, h1)


# speedup vs baseline: 1.1257x; 1.1257x over previous
import functools

import jax
import jax.numpy as jnp
from jax import lax
from jax.experimental import pallas as pl
from jax.experimental.pallas import tpu as pltpu
from jax.experimental.pallas import tpu_sc as plsc

N_NODES = 10000
N_EDGES = 160000
D = 256
DH = 128

NPAD = 10240
EPAD = 163840
CHUNK = 128
EPT = EPAD // 16
NCHUNKS = EPT // CHUNK
ROWS_PT = NPAD // 16


def _degrees_body(ei_ref, deg_ref, idx_v, zb_v, ones_v, hist_sh):
    c = lax.axis_index("c")
    s = lax.axis_index("s")

    z16 = jnp.zeros((16,), jnp.float32)
    o16 = jnp.ones((16,), jnp.float32)
    for j in range(CHUNK // 16):
        ones_v[pl.ds(16 * j, 16)] = o16

    def zfill(i, _):
        zb_v[pl.ds(16 * i, 16)] = z16
        return 0

    lax.fori_loop(0, ROWS_PT // 16, zfill, 0)

    pltpu.sync_copy(zb_v, hist_sh.at[pl.ds(s * ROWS_PT, ROWS_PT)])
    plsc.subcore_barrier()

    def chunk_step(k, _):
        base = s * EPT + k * CHUNK
        pltpu.sync_copy(ei_ref.at[c, pl.ds(base, CHUNK)], idx_v)
        pltpu.sync_copy(ones_v, hist_sh.at[idx_v], add=True)
        return 0

    lax.fori_loop(0, NCHUNKS, chunk_step, 0)
    plsc.subcore_barrier()

    pltpu.sync_copy(hist_sh.at[pl.ds(s * ROWS_PT, ROWS_PT)],
                    deg_ref.at[c, pl.ds(s * ROWS_PT, ROWS_PT)])


@functools.partial(
    pl.kernel,
    out_type=jax.ShapeDtypeStruct((2, NPAD), jnp.float32),
    mesh=plsc.VectorSubcoreMesh(core_axis_name="c", subcore_axis_name="s"),
    scratch_types=[
        pltpu.VMEM((CHUNK,), jnp.int32),
        pltpu.VMEM((ROWS_PT,), jnp.float32),
        pltpu.VMEM((CHUNK,), jnp.float32),
        pltpu.VMEM_SHARED((NPAD,), jnp.float32),
    ],
)
def _degrees(ei_ref, deg_ref, idx_v, zb_v, ones_v, hist_sh):
    _degrees_body(ei_ref, deg_ref, idx_v, zb_v, ones_v, hist_sh)


def _gs_body(h2_ref, gidx_ref, didx_ref, agg_ref,
             gi_a, di_a, di_b, rows_a, rows_b, sem_a, sem_b, agg_sh):
    c = lax.axis_index("c")
    s = lax.axis_index("s")

    z16 = jnp.zeros((16,), jnp.float32)

    def zrow(r, _):
        for j in range(DH // 16):
            rows_a[r, pl.ds(16 * j, 16)] = z16
        return 0

    lax.fori_loop(0, CHUNK, zrow, 0)

    def zcopy(i, _):
        pltpu.sync_copy(
            rows_a, agg_sh.at[pl.ds(s * ROWS_PT + i * CHUNK, CHUNK)])
        return 0

    lax.fori_loop(0, ROWS_PT // CHUNK, zcopy, 0)
    plsc.subcore_barrier()

    pltpu.sync_copy(gidx_ref.at[c, s], gi_a)

    def start_gather(k, rows_v, sem):
        pltpu.async_copy(h2_ref.at[gi_a.at[k]], rows_v, sem)

    def finish_chunk(k, di_v, rows_v, sem):
        base = s * EPT + k * CHUNK
        pltpu.sync_copy(didx_ref.at[pl.ds(base, CHUNK)], di_v)
        pltpu.make_async_copy(h2_ref.at[gi_a.at[k]], rows_v, sem).wait()
        pltpu.sync_copy(rows_v, agg_sh.at[di_v], add=True)

    start_gather(0, rows_a, sem_a)

    def pair_round(r, _):
        k = r * 2
        start_gather(k + 1, rows_b, sem_b)
        finish_chunk(k, di_a, rows_a, sem_a)

        @pl.when(k + 2 < NCHUNKS)
        def _():
            start_gather(k + 2, rows_a, sem_a)

        finish_chunk(k + 1, di_b, rows_b, sem_b)
        return 0

    lax.fori_loop(0, NCHUNKS // 2, pair_round, 0)
    plsc.subcore_barrier()

    pltpu.sync_copy(agg_sh.at[pl.ds(s * ROWS_PT, ROWS_PT)],
                    agg_ref.at[c, pl.ds(s * ROWS_PT, ROWS_PT)])


@functools.partial(
    pl.kernel,
    out_type=jax.ShapeDtypeStruct((2, NPAD, DH), jnp.float32),
    mesh=plsc.VectorSubcoreMesh(core_axis_name="c", subcore_axis_name="s"),
    scratch_types=[
        pltpu.VMEM((NCHUNKS, CHUNK), jnp.int32),
        pltpu.VMEM((CHUNK,), jnp.int32),
        pltpu.VMEM((CHUNK,), jnp.int32),
        pltpu.VMEM((CHUNK, DH), jnp.float32),
        pltpu.VMEM((CHUNK, DH), jnp.float32),
        pltpu.SemaphoreType.DMA,
        pltpu.SemaphoreType.DMA,
        pltpu.VMEM_SHARED((NPAD, DH), jnp.float32),
    ],
)
def _gather_scatter(h2_ref, gidx_ref, didx_ref, agg_ref,
                    gi_a, di_a, di_b, rows_a, rows_b, sem_a, sem_b, agg_sh):
    _gs_body(h2_ref, gidx_ref, didx_ref, agg_ref, gi_a, di_a, di_b,
             rows_a, rows_b, sem_a, sem_b, agg_sh)


def _matmul_scale_kernel(x_ref, w_ref, deg_ref, out_ref):
    deg = deg_ref[...]
    norm = jnp.where(deg > 0.0,
                     lax.rsqrt(jnp.maximum(deg, 1e-12)),
                     0.0)
    h = jnp.dot(x_ref[...], w_ref[...],
                preferred_element_type=jnp.float32) * norm
    out_ref[0] = h[:, :DH]
    out_ref[1] = h[:, DH:]


def _finalize_kernel(agg_ref, deg_ref, b_ref, out_ref):
    deg = deg_ref[...]
    norm = jnp.where(deg > 0.0,
                     lax.rsqrt(jnp.maximum(deg, 1e-12)),
                     0.0)
    agg = jnp.concatenate(
        [agg_ref[0, :N_NODES, :], agg_ref[1, :N_NODES, :]], axis=1)
    out_ref[...] = agg * norm + b_ref[...]


def kernel(inputs, edge_index, W, b):
    src = edge_index[0]
    dst = edge_index[1]

    pad = EPAD - N_EDGES
    pad_bin = jnp.full((pad,), NPAD - 1, jnp.int32)
    ei_deg = jnp.stack([jnp.concatenate([src, pad_bin]),
                        jnp.concatenate([dst, pad_bin])])
    pad_zero = jnp.zeros((pad,), jnp.int32)
    gidx = jnp.stack([jnp.concatenate([src, pad_zero]),
                      jnp.concatenate([src + N_NODES, pad_zero])])
    gidx = gidx.reshape(2, 16, NCHUNKS, CHUNK)
    didx = jnp.concatenate([dst, pad_bin])

    deg = _degrees(ei_deg)

    h2 = pl.pallas_call(
        _matmul_scale_kernel,
        out_shape=jax.ShapeDtypeStruct((2, N_NODES, DH), jnp.float32),
    )(inputs, W, deg[0, :N_NODES].reshape(N_NODES, 1))
    h2 = h2.reshape(2 * N_NODES, DH)

    agg = _gather_scatter(h2, gidx, didx)

    h1 = pl.pallas_call(
        _finalize_kernel,
        out_shape=jax.ShapeDtypeStruct((N_NODES, D), jnp.float32),
    )(agg, deg[1, :N_NODES].reshape(N_NODES, 1), b.reshape(1, D))
    return (h1, h1)
